# speedup vs baseline: 8.6435x; 8.6435x over previous
import jax
import jax.numpy as jnp
from jax import lax
from jax.experimental import pallas as pl
from jax.experimental.pallas import tpu as pltpu
from jax.experimental.pallas import tpu_sc as plsc

N_NODES = 10000
N_EDGES = 160000
D = 256
HALF = 128
N_PAD = 10240
NC, NS = 2, 16
NW = NC * NS
EPW = 5120
E_PAD = EPW * NW
CHUNK = 128
NCHUNK = EPW // CHUNK
ROWS_PT = N_PAD // NS

_sc_mesh = plsc.VectorSubcoreMesh(core_axis_name="c", subcore_axis_name="s")


def _deg_body(dsts_hbm, deg_hbm, dst_v, ones_v, acc_sh):
    c = lax.axis_index("c")
    s = lax.axis_index("s")

    @pl.when(c == 0)
    def _():
        def fill(k, carry):
            ones_v[k, :] = jnp.full((16,), 1.0, jnp.float32)
            return carry

        lax.fori_loop(0, CHUNK, fill, 0)
        for r in range(ROWS_PT // CHUNK):
            pltpu.sync_copy(ones_v, acc_sh.at[pl.ds(s * ROWS_PT + r * CHUNK, CHUNK)])
        pltpu.sync_copy(dsts_hbm.at[pl.ds(s * 2, 2)], dst_v)
        plsc.subcore_barrier()

        for u in range(2):
            def step(j, carry):
                pltpu.sync_copy(ones_v, acc_sh.at[dst_v.at[u, j]], add=True)
                return carry

            lax.fori_loop(0, NCHUNK, step, 0)
        plsc.subcore_barrier()
        pltpu.sync_copy(acc_sh.at[pl.ds(s * ROWS_PT, ROWS_PT)],
                        deg_hbm.at[pl.ds(s * ROWS_PT, ROWS_PT)])


_deg_kernel = pl.kernel(
    _deg_body,
    out_type=jax.ShapeDtypeStruct((N_PAD, 16), jnp.float32),
    mesh=_sc_mesh,
    scratch_types=[
        pltpu.VMEM((2, NCHUNK, CHUNK), jnp.int32),
        pltpu.VMEM((CHUNK, 16), jnp.float32),
        pltpu.VMEM_SHARED((N_PAD, 16), jnp.float32),
    ],
)


def _agg_body(h_hbm, srcs_hbm, dsts_hbm, out_hbm, src_v, dst_v, rows_v, acc_sh):
    c = lax.axis_index("c")
    s = lax.axis_index("s")
    w = s * NC + c
    off = c * N_PAD
    pltpu.sync_copy(srcs_hbm.at[w], src_v)
    pltpu.sync_copy(dsts_hbm.at[w], dst_v)

    def add_off(t, carry):
        src_v[pl.ds(t * 16, 16)] = src_v[pl.ds(t * 16, 16)] + off
        return carry

    lax.fori_loop(0, EPW // 16, add_off, 0)
    pltpu.sync_copy(h_hbm.at[pl.ds(off + s * ROWS_PT, ROWS_PT)],
                    acc_sh.at[pl.ds(s * ROWS_PT, ROWS_PT)])
    plsc.subcore_barrier()

    def step(j, carry):
        pltpu.sync_copy(h_hbm.at[src_v.at[pl.ds(j * CHUNK, CHUNK)]], rows_v)
        pltpu.sync_copy(rows_v, acc_sh.at[dst_v.at[j]], add=True)
        return carry

    lax.fori_loop(0, NCHUNK, step, 0)
    plsc.subcore_barrier()
    pltpu.sync_copy(acc_sh.at[pl.ds(s * ROWS_PT, ROWS_PT)],
                    out_hbm.at[pl.ds(off + s * ROWS_PT, ROWS_PT)])


_agg_kernel = pl.kernel(
    _agg_body,
    out_type=jax.ShapeDtypeStruct((2 * N_PAD, HALF), jnp.float32),
    mesh=_sc_mesh,
    scratch_types=[
        pltpu.VMEM((EPW,), jnp.int32),
        pltpu.VMEM((NCHUNK, CHUNK), jnp.int32),
        pltpu.VMEM((CHUNK, HALF), jnp.float32),
        pltpu.VMEM_SHARED((N_PAD, HALF), jnp.float32),
    ],
)


RB = 1280
NRB = N_PAD // RB


def _mm1_body(x_ref, deg_ref, w_ref, o_ref):
    dis = lax.rsqrt(deg_ref[:, 0:1])
    o_ref[...] = jnp.dot(x_ref[...] * dis, w_ref[...],
                         preferred_element_type=jnp.float32)


_mm1 = pl.pallas_call(
    _mm1_body,
    grid=(NRB, 2),
    in_specs=[
        pl.BlockSpec((RB, D), lambda i, j: (i, 0)),
        pl.BlockSpec((RB, 16), lambda i, j: (i, 0)),
        pl.BlockSpec((D, HALF), lambda i, j: (0, j)),
    ],
    out_specs=pl.BlockSpec((RB, HALF), lambda i, j: (j * NRB + i, 0)),
    out_shape=jax.ShapeDtypeStruct((2 * N_PAD, HALF), jnp.float32),
)


def _mm2_body(agg_a_ref, agg_b_ref, deg_ref, b_ref, w_ref, o_ref):
    dis = lax.rsqrt(deg_ref[:, 0:1])
    o = jnp.concatenate([agg_a_ref[...], agg_b_ref[...]], axis=1) * dis + b_ref[...]
    h = jnp.maximum(o, 0.0) * dis
    o_ref[...] = jnp.dot(h, w_ref[...], preferred_element_type=jnp.float32)


_mm2 = pl.pallas_call(
    _mm2_body,
    grid=(NRB, 2),
    in_specs=[
        pl.BlockSpec((RB, HALF), lambda i, j: (i, 0)),
        pl.BlockSpec((RB, HALF), lambda i, j: (i + NRB, 0)),
        pl.BlockSpec((RB, 16), lambda i, j: (i, 0)),
        pl.BlockSpec((1, D), lambda i, j: (0, 0)),
        pl.BlockSpec((D, HALF), lambda i, j: (0, j)),
    ],
    out_specs=pl.BlockSpec((RB, HALF), lambda i, j: (j * NRB + i, 0)),
    out_shape=jax.ShapeDtypeStruct((2 * N_PAD, HALF), jnp.float32),
)


def _fin_body(agg_a_ref, agg_b_ref, deg_ref, b_ref, o_ref):
    dis = lax.rsqrt(deg_ref[:, 0:1])
    o = jnp.concatenate([agg_a_ref[...], agg_b_ref[...]], axis=1) * dis + b_ref[...]
    m = jnp.max(o, axis=1, keepdims=True)
    e = jnp.exp(o - m)
    o_ref[...] = e / jnp.sum(e, axis=1, keepdims=True)


_fin = pl.pallas_call(
    _fin_body,
    grid=(NRB,),
    in_specs=[
        pl.BlockSpec((RB, HALF), lambda i: (i, 0)),
        pl.BlockSpec((RB, HALF), lambda i: (i + NRB, 0)),
        pl.BlockSpec((RB, 16), lambda i: (i, 0)),
        pl.BlockSpec((1, D), lambda i: (0, 0)),
    ],
    out_specs=pl.BlockSpec((RB, D), lambda i: (i, 0)),
    out_shape=jax.ShapeDtypeStruct((N_PAD, D), jnp.float32),
)


def kernel(feature, edge_index, W0, b0, W1, b1):
    x = jnp.pad(feature.astype(jnp.float32), ((0, N_PAD - N_NODES), (0, 0)))
    src = edge_index[0].astype(jnp.int32)
    dst = edge_index[1].astype(jnp.int32)
    pad_e = E_PAD - N_EDGES
    fill = jnp.full((pad_e,), N_NODES, jnp.int32)
    src_p = jnp.concatenate([src, fill]).reshape(NW, EPW)
    dst_p = jnp.concatenate([dst, fill]).reshape(NW, NCHUNK, CHUNK)

    deg = _deg_kernel(dst_p)
    h1 = _mm1(x, deg, W0)
    agg1 = _agg_kernel(h1, src_p, dst_p)
    h2 = _mm2(agg1, agg1, deg, b0.reshape(1, D), W1)
    agg2 = _agg_kernel(h2, src_p, dst_p)
    out = _fin(agg2, agg2, deg, b1.reshape(1, D))
    return out[:N_NODES]


# speedup vs baseline: 8.8441x; 1.0232x over previous
import jax
import jax.numpy as jnp
from jax import lax
from jax.experimental import pallas as pl
from jax.experimental.pallas import tpu as pltpu
from jax.experimental.pallas import tpu_sc as plsc

N_NODES = 10000
N_EDGES = 160000
D = 256
HALF = 128
N_PAD = 10240
NC, NS = 2, 16
NW = NC * NS
EPW = 5120
E_PAD = EPW * NW
CHUNK = 128
NCHUNK = EPW // CHUNK
ROWS_PT = N_PAD // NS

_sc_mesh = plsc.VectorSubcoreMesh(core_axis_name="c", subcore_axis_name="s")


def _deg_body(dsts_hbm, deg_hbm, dst_v, ones_v, acc_sh):
    c = lax.axis_index("c")
    s = lax.axis_index("s")

    @pl.when(c == 0)
    def _():
        def fill(k, carry):
            ones_v[k, :] = jnp.full((16,), 1.0, jnp.float32)
            return carry

        lax.fori_loop(0, CHUNK, fill, 0)
        for r in range(ROWS_PT // CHUNK):
            pltpu.sync_copy(ones_v, acc_sh.at[pl.ds(s * ROWS_PT + r * CHUNK, CHUNK)])
        pltpu.sync_copy(dsts_hbm.at[pl.ds(s * 2, 2)], dst_v)
        plsc.subcore_barrier()

        for u in range(2):
            def step(j, carry):
                pltpu.sync_copy(ones_v, acc_sh.at[dst_v.at[u, j]], add=True)
                return carry

            lax.fori_loop(0, NCHUNK, step, 0)
        plsc.subcore_barrier()
        pltpu.sync_copy(acc_sh.at[pl.ds(s * ROWS_PT, ROWS_PT)],
                        deg_hbm.at[pl.ds(s * ROWS_PT, ROWS_PT)])


_deg_kernel = pl.kernel(
    _deg_body,
    out_type=jax.ShapeDtypeStruct((N_PAD, 16), jnp.float32),
    mesh=_sc_mesh,
    scratch_types=[
        pltpu.VMEM((2, NCHUNK, CHUNK), jnp.int32),
        pltpu.VMEM((CHUNK, 16), jnp.float32),
        pltpu.VMEM_SHARED((N_PAD, 16), jnp.float32),
    ],
)


NBUF = 2


def _agg_body(h_hbm, srcs_hbm, dsts_hbm, out_hbm, src_v, dst_v, rows_v, acc_sh,
              sem_g, sem_s):
    c = lax.axis_index("c")
    s = lax.axis_index("s")
    w = s * NC + c
    off = c * N_PAD
    pltpu.sync_copy(srcs_hbm.at[w], src_v)
    pltpu.sync_copy(dsts_hbm.at[w], dst_v)

    def add_off(t, carry):
        src_v[pl.ds(t * 16, 16)] = src_v[pl.ds(t * 16, 16)] + off
        return carry

    lax.fori_loop(0, EPW // 16, add_off, 0)
    pltpu.sync_copy(h_hbm.at[pl.ds(off + s * ROWS_PT, ROWS_PT)],
                    acc_sh.at[pl.ds(s * ROWS_PT, ROWS_PT)])
    plsc.subcore_barrier()

    def group(k, carry):
        dg = []
        for b in range(NBUF):
            t = k * NBUF + b
            dg.append(pltpu.async_copy(
                h_hbm.at[src_v.at[pl.ds(t * CHUNK, CHUNK)]],
                rows_v.at[b], sem_g.at[b]))
        ds = []
        for b in range(NBUF):
            t = k * NBUF + b
            dg[b].wait()
            ds.append(pltpu.async_copy(
                rows_v.at[b], acc_sh.at[dst_v.at[t]], sem_s.at[b], add=True))
        for b in range(NBUF):
            ds[b].wait()
        return carry

    lax.fori_loop(0, NCHUNK // NBUF, group, 0)
    plsc.subcore_barrier()
    pltpu.sync_copy(acc_sh.at[pl.ds(s * ROWS_PT, ROWS_PT)],
                    out_hbm.at[pl.ds(off + s * ROWS_PT, ROWS_PT)])


_agg_kernel = pl.kernel(
    _agg_body,
    out_type=jax.ShapeDtypeStruct((2 * N_PAD, HALF), jnp.float32),
    mesh=_sc_mesh,
    scratch_types=[
        pltpu.VMEM((EPW,), jnp.int32),
        pltpu.VMEM((NCHUNK, CHUNK), jnp.int32),
        pltpu.VMEM((NBUF, CHUNK, HALF), jnp.float32),
        pltpu.VMEM_SHARED((N_PAD, HALF), jnp.float32),
        pltpu.SemaphoreType.DMA((NBUF,)),
        pltpu.SemaphoreType.DMA((NBUF,)),
    ],
)


RB = 1280
NRB = N_PAD // RB


def _mm1_body(x_ref, deg_ref, w_ref, o_ref):
    dis = lax.rsqrt(deg_ref[:, 0:1])
    o_ref[...] = jnp.dot(x_ref[...] * dis, w_ref[...],
                         preferred_element_type=jnp.float32)


_mm1 = pl.pallas_call(
    _mm1_body,
    grid=(NRB, 2),
    in_specs=[
        pl.BlockSpec((RB, D), lambda i, j: (i, 0)),
        pl.BlockSpec((RB, 16), lambda i, j: (i, 0)),
        pl.BlockSpec((D, HALF), lambda i, j: (0, j)),
    ],
    out_specs=pl.BlockSpec((RB, HALF), lambda i, j: (j * NRB + i, 0)),
    out_shape=jax.ShapeDtypeStruct((2 * N_PAD, HALF), jnp.float32),
)


def _mm2_body(agg_a_ref, agg_b_ref, deg_ref, b_ref, w_ref, o_ref):
    dis = lax.rsqrt(deg_ref[:, 0:1])
    o = jnp.concatenate([agg_a_ref[...], agg_b_ref[...]], axis=1) * dis + b_ref[...]
    h = jnp.maximum(o, 0.0) * dis
    o_ref[...] = jnp.dot(h, w_ref[...], preferred_element_type=jnp.float32)


_mm2 = pl.pallas_call(
    _mm2_body,
    grid=(NRB, 2),
    in_specs=[
        pl.BlockSpec((RB, HALF), lambda i, j: (i, 0)),
        pl.BlockSpec((RB, HALF), lambda i, j: (i + NRB, 0)),
        pl.BlockSpec((RB, 16), lambda i, j: (i, 0)),
        pl.BlockSpec((1, D), lambda i, j: (0, 0)),
        pl.BlockSpec((D, HALF), lambda i, j: (0, j)),
    ],
    out_specs=pl.BlockSpec((RB, HALF), lambda i, j: (j * NRB + i, 0)),
    out_shape=jax.ShapeDtypeStruct((2 * N_PAD, HALF), jnp.float32),
)


def _fin_body(agg_a_ref, agg_b_ref, deg_ref, b_ref, o_ref):
    dis = lax.rsqrt(deg_ref[:, 0:1])
    o = jnp.concatenate([agg_a_ref[...], agg_b_ref[...]], axis=1) * dis + b_ref[...]
    m = jnp.max(o, axis=1, keepdims=True)
    e = jnp.exp(o - m)
    o_ref[...] = e / jnp.sum(e, axis=1, keepdims=True)


_fin = pl.pallas_call(
    _fin_body,
    grid=(NRB,),
    in_specs=[
        pl.BlockSpec((RB, HALF), lambda i: (i, 0)),
        pl.BlockSpec((RB, HALF), lambda i: (i + NRB, 0)),
        pl.BlockSpec((RB, 16), lambda i: (i, 0)),
        pl.BlockSpec((1, D), lambda i: (0, 0)),
    ],
    out_specs=pl.BlockSpec((RB, D), lambda i: (i, 0)),
    out_shape=jax.ShapeDtypeStruct((N_PAD, D), jnp.float32),
)


def kernel(feature, edge_index, W0, b0, W1, b1):
    x = jnp.pad(feature.astype(jnp.float32), ((0, N_PAD - N_NODES), (0, 0)))
    src = edge_index[0].astype(jnp.int32)
    dst = edge_index[1].astype(jnp.int32)
    pad_e = E_PAD - N_EDGES
    fill = jnp.full((pad_e,), N_NODES, jnp.int32)
    src_p = jnp.concatenate([src, fill]).reshape(NW, EPW)
    dst_p = jnp.concatenate([dst, fill]).reshape(NW, NCHUNK, CHUNK)

    deg = _deg_kernel(dst_p)
    h1 = _mm1(x, deg, W0)
    agg1 = _agg_kernel(h1, src_p, dst_p)
    h2 = _mm2(agg1, agg1, deg, b0.reshape(1, D), W1)
    agg2 = _agg_kernel(h2, src_p, dst_p)
    out = _fin(agg2, agg2, deg, b1.reshape(1, D))
    return out[:N_NODES]
